# Initial kernel scaffold; baseline (speedup 1.0000x reference)
import jax
import jax.numpy as jnp
from jax.experimental import pallas as pl


def kernel(x, W1, W2, W3):
    raise NotImplementedError("write your pallas kernel here")


# speedup vs baseline: 1.3129x; 1.3129x over previous
import jax
import jax.numpy as jnp
from jax.experimental import pallas as pl
from jax.experimental.pallas import tpu as pltpu

F32 = jnp.float32
_VMEM_LIMIT = 56 * 1024 * 1024
_INTERPRET = False


def _cparams(*sems):
    return pltpu.CompilerParams(
        dimension_semantics=tuple(sems),
        vmem_limit_bytes=_VMEM_LIMIT,
    )


def _gemv_kernel(h_ref, w_ref, z_ref, *, relu):
    h = h_ref[...]
    if relu:
        h = jnp.maximum(h, 0.0)
    z_ref[...] = jax.lax.dot_general(
        h, w_ref[...], (((1,), (1,)), ((), ())),
        preferred_element_type=F32)


def _gemv(h, W, bj, relu):
    import functools
    J, K = W.shape
    return pl.pallas_call(
        functools.partial(_gemv_kernel, relu=relu),
        grid=(J // bj,),
        in_specs=[pl.BlockSpec((1, K), lambda j: (0, 0)),
                  pl.BlockSpec((bj, K), lambda j: (j, 0))],
        out_specs=pl.BlockSpec((1, bj), lambda j: (0, j)),
        out_shape=jax.ShapeDtypeStruct((1, J), F32),
        compiler_params=_cparams("arbitrary"),
        name="gemv",
        interpret=_INTERPRET,
    )(h, W)


def _transpose_kernel(w_ref, o_ref):
    o_ref[...] = w_ref[...].T


def _transpose(W, b=512):
    R, C = W.shape
    return pl.pallas_call(
        _transpose_kernel,
        grid=(R // b, C // b),
        in_specs=[pl.BlockSpec((b, b), lambda r, c: (r, c))],
        out_specs=pl.BlockSpec((b, b), lambda r, c: (c, r)),
        out_shape=jax.ShapeDtypeStruct((C, R), F32),
        compiler_params=_cparams("arbitrary", "arbitrary"),
        name="transpose",
        interpret=_INTERPRET,
    )(W)


def _diag_kernel(z1_ref, z2_ref, o1_ref, o2_ref, *, br, n):
    r = pl.program_id(0)
    rows = jax.lax.broadcasted_iota(jnp.int32, (br, n), 0) + r * br
    cols = jax.lax.broadcasted_iota(jnp.int32, (br, n), 1)
    eq = rows == cols
    o1_ref[...] = jnp.where(eq, (z1_ref[...] > 0).astype(F32), 0.0)
    o2_ref[...] = jnp.where(eq, (z2_ref[...] > 0).astype(F32), 0.0)


def _diag_pair(z1, z2, br=512):
    import functools
    n = z1.shape[1]
    out = jax.ShapeDtypeStruct((n, n), F32)
    return pl.pallas_call(
        functools.partial(_diag_kernel, br=br, n=n),
        grid=(n // br,),
        in_specs=[pl.BlockSpec((1, n), lambda r: (0, 0)),
                  pl.BlockSpec((1, n), lambda r: (0, 0))],
        out_specs=[pl.BlockSpec((br, n), lambda r: (r, 0)),
                   pl.BlockSpec((br, n), lambda r: (r, 0))],
        out_shape=[out, out],
        compiler_params=_cparams("arbitrary"),
        name="diag_pair",
        interpret=_INTERPRET,
    )(z1, z2)


def _mm1_kernel(a_ref, z_ref, b_ref, o_ref):
    scale = (z_ref[...] > 0).astype(F32)
    a = a_ref[...] * scale
    o_ref[...] = jnp.dot(a, b_ref[...], preferred_element_type=F32)


def _mm1(A, z, B, bi, bj):
    M, K = A.shape
    _, N = B.shape
    return pl.pallas_call(
        _mm1_kernel,
        grid=(M // bi, N // bj),
        in_specs=[pl.BlockSpec((bi, K), lambda i, j: (i, 0)),
                  pl.BlockSpec((1, K), lambda i, j: (0, 0)),
                  pl.BlockSpec((K, bj), lambda i, j: (0, j))],
        out_specs=pl.BlockSpec((bi, bj), lambda i, j: (i, j)),
        out_shape=jax.ShapeDtypeStruct((M, N), F32),
        compiler_params=_cparams("arbitrary", "arbitrary"),
        name="scaled_mm1",
        interpret=_INTERPRET,
    )(A, z, B)


def _mm2_kernel(a_ref, z_ref, b_ref, o_ref, eye_ref, *, bi, bl):
    i = pl.program_id(0)
    l = pl.program_id(1)
    scale = (z_ref[...] > 0).astype(F32)
    a = a_ref[...] * scale
    o_ref[...] = jnp.dot(a, b_ref[...], preferred_element_type=F32)
    rows = jax.lax.broadcasted_iota(jnp.int32, (bi, bl), 0) + i * bi
    cols = jax.lax.broadcasted_iota(jnp.int32, (bi, bl), 1) + l * bl
    eye_ref[...] = jnp.where(rows == cols, 1.0, 0.0).astype(F32)


def _mm2(A, z, B, bi, bl):
    import functools
    M, K = A.shape
    _, N = B.shape
    return pl.pallas_call(
        functools.partial(_mm2_kernel, bi=bi, bl=bl),
        grid=(M // bi, N // bl),
        in_specs=[pl.BlockSpec((bi, K), lambda i, l: (i, 0)),
                  pl.BlockSpec((1, K), lambda i, l: (0, 0)),
                  pl.BlockSpec((K, bl), lambda i, l: (0, l))],
        out_specs=[pl.BlockSpec((bi, bl), lambda i, l: (i, l)),
                   pl.BlockSpec((bi, bl), lambda i, l: (i, l))],
        out_shape=[jax.ShapeDtypeStruct((M, N), F32),
                   jax.ShapeDtypeStruct((M, N), F32)],
        compiler_params=_cparams("arbitrary", "arbitrary"),
        name="scaled_mm2_eye",
        interpret=_INTERPRET,
    )(A, z, B)


def kernel(x, W1, W2, W3):
    z1 = _gemv(x, W1, 512, relu=False)
    z2 = _gemv(z1, W2, 512, relu=True)
    out = _gemv(z2, W3, 512, relu=True)

    W1T = _transpose(W1)
    W2T = _transpose(W2)
    W3T = _transpose(W3)
    D1, D2 = _diag_pair(z1, z2)

    T1 = _mm1(W1T, z1, W2T, bi=1024, bj=256)
    DJM, EYE = _mm2(T1, z2, W3T, bi=1024, bl=256)

    return (out, DJM, W1T, D1, W2T, D2, W3T, EYE)
